# baseline (speedup 1.0000x reference)
import functools

import jax
import jax.numpy as jnp
import numpy as _np
from jax import lax
from jax.experimental import pallas as pl
from jax.experimental.pallas import tpu as pltpu
from jax.experimental.pallas import tpu_sc as plsc

NN = 10000
DD = 128
EE = 320000
NC = 2
NS = 16
NWK = NC * NS
CH = 128
EPAD = 327680
NROW = EPAD // CH
RPW = NROW // NWK
RPT = NROW // NS
NP = 10240
NPT = NP // NS
ROWS_PT = NN // NS

_mesh = plsc.VectorSubcoreMesh(core_axis_name="c", subcore_axis_name="s")


def _rsqrt_nr(x):
    i = plsc.bitcast(x, jnp.int32)
    i = jnp.int32(0x5F3759DF) - (i >> 1)
    y = plsc.bitcast(i, jnp.float32)
    for _ in range(4):
        y = y * (jnp.float32(1.5) - jnp.float32(0.5) * x * y * y)
    return y


@functools.partial(
    pl.kernel,
    out_type=jax.ShapeDtypeStruct((NROW, CH), jnp.float32),
    mesh=_mesh,
    scratch_types=[
        pltpu.VMEM((RPT, CH), jnp.int32),
        pltpu.VMEM((RPT, CH), jnp.float32),
        pltpu.VMEM((RPW, CH), jnp.int32),
        pltpu.VMEM((RPW, CH), jnp.float32),
        pltpu.VMEM((NP,), jnp.float32),
        pltpu.VMEM((NPT,), jnp.float32),
        pltpu.VMEM_SHARED((NP,), jnp.float32),
    ],
    compiler_params=pltpu.CompilerParams(needs_layout_passes=False),
)
def _norm_kernel(src_hbm, dst_hbm, ea_hbm, norm_hbm,
                 dst_v, ea_v, src_v, norm_v, tab_v, z_v, deg_sh):
    c = lax.axis_index("c")
    s = lax.axis_index("s")
    wid = c * NS + s

    for i in range(NPT // 16):
        z_v[pl.ds(i * 16, 16)] = jnp.zeros((16,), jnp.float32)
    pltpu.sync_copy(z_v, deg_sh.at[pl.ds(s * NPT, NPT)])
    plsc.subcore_barrier()

    pltpu.sync_copy(dst_hbm.at[pl.ds(s * RPT, RPT)], dst_v)
    pltpu.sync_copy(ea_hbm.at[pl.ds(s * RPT, RPT)], ea_v)

    def deg_body(j, carry):
        pltpu.sync_copy(ea_v.at[j], deg_sh.at[dst_v.at[j]], add=True)
        return carry

    lax.fori_loop(0, RPT, deg_body, 0)
    plsc.subcore_barrier()

    pltpu.sync_copy(deg_sh, tab_v)
    for i in range(NP // 16):
        d = tab_v[pl.ds(i * 16, 16)]
        ok = d > jnp.float32(0.0)
        y = _rsqrt_nr(jnp.where(ok, d, jnp.float32(1.0)))
        tab_v[pl.ds(i * 16, 16)] = jnp.where(ok, y, jnp.float32(0.0))

    pltpu.sync_copy(src_hbm.at[pl.ds(wid * RPW, RPW)], src_v)
    pltpu.sync_copy(dst_hbm.at[pl.ds(wid * RPW, RPW)], dst_v.at[pl.ds(0, RPW)])
    pltpu.sync_copy(ea_hbm.at[pl.ds(wid * RPW, RPW)], ea_v.at[pl.ds(0, RPW)])

    def norm_body(j, carry):
        for g in range(CH // 16):
            sl = pl.ds(g * 16, 16)
            s16 = src_v[j, sl]
            d16 = dst_v[j, sl]
            a16 = ea_v[j, sl]
            n16 = (plsc.load_gather(tab_v, [s16]) * a16
                   * plsc.load_gather(tab_v, [d16]))
            norm_v[j, sl] = n16
        return carry

    lax.fori_loop(0, RPW, norm_body, 0)
    pltpu.sync_copy(norm_v, norm_hbm.at[pl.ds(wid * RPW, RPW)])


_GDN = lax.GatherDimensionNumbers(offset_dims=(), collapsed_slice_dims=(0,),
                                  start_index_map=(0,))


def _splat(v, u):
    return lax.gather(v, jnp.full((16, 1), u, jnp.int32), _GDN, (1,),
                      mode=lax.GatherScatterMode.PROMISE_IN_BOUNDS)

NB = 4
CHP = 32
NROWP = EPAD // CHP
RPWP = NROWP // NWK
GG = 16
NGRP = RPWP // GG
ZCP = 640 // CHP


@functools.partial(
    pl.kernel,
    out_type=jax.ShapeDtypeStruct((NC, NN, DD), jnp.float32),
    mesh=_mesh,
    scratch_types=[
        pltpu.VMEM((2, GG, CHP), jnp.int32),
        pltpu.VMEM((2, GG, CHP), jnp.int32),
        pltpu.VMEM((2, GG, CHP), jnp.float32),
        pltpu.VMEM((NB, CHP, DD), jnp.float32),
        pltpu.VMEM_SHARED((NN, DD), jnp.float32),
        [pltpu.SemaphoreType.DMA] * 2,
        [pltpu.SemaphoreType.DMA] * NB,
        [pltpu.SemaphoreType.DMA] * NB,
    ],
    compiler_params=pltpu.CompilerParams(needs_layout_passes=False),
)
def _prop_kernel(h_hbm, src_hbm, dst_hbm, norm_hbm, out_hbm,
                 src_v, dst_v, norm_v, rows_v, acc_sh, esem, gsem, ssem):
    c = lax.axis_index("c")
    s = lax.axis_index("s")
    wid = c * NS + s
    base = wid * RPWP

    def stage_issue(grp, buf):
        rows = pl.ds(base + grp * GG, GG)
        pltpu.async_copy(src_hbm.at[rows], src_v.at[buf], esem[buf])
        pltpu.async_copy(dst_hbm.at[rows], dst_v.at[buf], esem[buf])
        pltpu.async_copy(norm_hbm.at[rows], norm_v.at[buf], esem[buf])

    def stage_wait(grp, buf):
        rows = pl.ds(base + grp * GG, GG)
        pltpu.make_async_copy(src_hbm.at[rows], src_v.at[buf],
                              esem[buf]).wait()
        pltpu.make_async_copy(dst_hbm.at[rows], dst_v.at[buf],
                              esem[buf]).wait()
        pltpu.make_async_copy(norm_hbm.at[rows], norm_v.at[buf],
                              esem[buf]).wait()

    def zero_body(r, zc):
        for g in range(DD // 16):
            rows_v[0, r, pl.ds(g * 16, 16)] = jnp.zeros((16,), jnp.float32)
        return zc

    lax.fori_loop(0, CHP, zero_body, 0)
    for t in range(ZCP):
        pltpu.sync_copy(rows_v.at[0],
                        acc_sh.at[pl.ds(s * 624 + t * CHP, CHP)])
    plsc.subcore_barrier()

    stage_issue(0, 0)
    stage_issue(1, 1)
    stage_wait(0, 0)
    for b in range(NB):
        pltpu.async_copy(h_hbm.at[src_v.at[0, b]], rows_v.at[b], gsem[b])

    def scale_chunk(b, gbuf, r):
        def half_body(i, ecarry):
            norm16 = norm_v[gbuf, r, pl.ds(i * 16, 16)]
            for u in range(16):
                e = i * 16 + u
                spl = _splat(norm16, u)
                for g in range(DD // 16):
                    sl = pl.ds(g * 16, 16)
                    rows_v[b, e, sl] = rows_v[b, e, sl] * spl
            return ecarry

        lax.fori_loop(0, CHP // 16, half_body, 0)

    def pair_body(i2, carry):
        for pp in range(2):
            grp = i2 * 2 + pp
            for r in range(GG):
                k = grp * GG + r
                b = r % NB

                if r == 1:
                    if pp == 0:
                        cond = i2 >= 1
                    else:
                        cond = i2 <= NGRP // 2 - 2

                    @pl.when(cond)
                    def _(grp=grp, pp=pp):
                        stage_issue(grp + 1, (pp + 1) % 2)

                if r == GG - NB:
                    cond = (i2 <= NGRP // 2 - 2) if pp == 1 else (i2 >= 0)

                    @pl.when(cond)
                    def _(grp=grp, pp=pp):
                        stage_wait(grp + 1, (pp + 1) % 2)

                pltpu.make_async_copy(h_hbm.at[src_v.at[pp, r]],
                                      rows_v.at[b], gsem[b]).wait()
                scale_chunk(b, pp, r)
                pltpu.async_copy(rows_v.at[b], acc_sh.at[dst_v.at[pp, r]],
                                 ssem[b], add=True)

                pb = (b - 1) % NB
                rn = (r + NB - 1) % GG
                pn = (pp + (1 if r + NB - 1 >= GG else 0)) % 2
                rm = (r - 1) % GG
                pm = (pp - (1 if r == 0 else 0)) % 2
                cond = jnp.logical_and(k >= 1, k <= RPWP - NB)

                @pl.when(cond)
                def _(pb=pb, rn=rn, pn=pn, rm=rm, pm=pm):
                    pltpu.make_async_copy(rows_v.at[pb],
                                          acc_sh.at[dst_v.at[pm, rm]],
                                          ssem[pb]).wait()
                    pltpu.async_copy(h_hbm.at[src_v.at[pn, rn]],
                                     rows_v.at[pb], gsem[pb])
        return carry

    lax.fori_loop(0, NGRP // 2, pair_body, 0)

    for b in range(NB):
        pltpu.make_async_copy(
            rows_v.at[b],
            acc_sh.at[dst_v.at[(NGRP - 1) % 2, GG - NB + b]],
            ssem[b]).wait()
    plsc.subcore_barrier()

    for t in range(ZCP):
        r0 = s * 624 + t * CHP
        pltpu.sync_copy(acc_sh.at[pl.ds(r0, CHP)], rows_v.at[0])
        pltpu.sync_copy(rows_v.at[0], out_hbm.at[c, pl.ds(r0, CHP)])


_BLK = 2000
_NBLK = NN // _BLK


def _vspec(d=DD):
    return pl.BlockSpec((_BLK, d), lambda i: (i, 0))


def _wspec(fi, fo):
    return pl.BlockSpec((fi, fo), lambda i: (0, 0))


def _tc_first(h, pa, pb, w0, w1, fo):
    def body(h_r, pa_r, pb_r, w0_r, w1_r, p_r, acc_r):
        p = pa_r[...] + pb_r[...]
        p_r[...] = p
        acc_r[...] = (jnp.dot(h_r[...], w0_r[...],
                              preferred_element_type=jnp.float32)
                      + jnp.dot(p, w1_r[...],
                                preferred_element_type=jnp.float32))

    return pl.pallas_call(
        body,
        grid=(_NBLK,),
        in_specs=[_vspec(), _vspec(), _vspec(), _wspec(DD, fo), _wspec(DD, fo)],
        out_specs=[_vspec(), _vspec(fo)],
        out_shape=[jax.ShapeDtypeStruct((NN, DD), jnp.float32),
                   jax.ShapeDtypeStruct((NN, fo), jnp.float32)],
    )(h, pa, pb, w0, w1)


def _tc_mid(acc, pa, pb, w, fo):
    def body(acc_r, pa_r, pb_r, w_r, p_r, out_r):
        p = pa_r[...] + pb_r[...]
        p_r[...] = p
        out_r[...] = acc_r[...] + jnp.dot(p, w_r[...],
                                          preferred_element_type=jnp.float32)

    return pl.pallas_call(
        body,
        grid=(_NBLK,),
        in_specs=[_vspec(fo), _vspec(), _vspec(), _wspec(DD, fo)],
        out_specs=[_vspec(), _vspec(fo)],
        out_shape=[jax.ShapeDtypeStruct((NN, DD), jnp.float32),
                   jax.ShapeDtypeStruct((NN, fo), jnp.float32)],
    )(acc, pa, pb, w)


def _tc_last(acc, pa, pb, w, b, fo, act):
    def body(acc_r, pa_r, pb_r, w_r, b_r, out_r):
        p = pa_r[...] + pb_r[...]
        o = acc_r[...] + jnp.dot(p, w_r[...],
                                 preferred_element_type=jnp.float32) + b_r[...]
        if act == "relu":
            out_r[...] = jnp.maximum(o, 0.0)
        else:
            m = jnp.max(o, axis=1, keepdims=True)
            z = o - m
            lse = jnp.log(jnp.sum(jnp.exp(z), axis=1, keepdims=True))
            out_r[...] = z - lse

    return pl.pallas_call(
        body,
        grid=(_NBLK,),
        in_specs=[_vspec(fo), _vspec(), _vspec(), _wspec(DD, fo),
                  pl.BlockSpec((1, fo), lambda i: (0, 0))],
        out_specs=_vspec(fo),
        out_shape=jax.ShapeDtypeStruct((NN, fo), jnp.float32),
    )(acc, pa, pb, w, b.reshape(1, fo))


def kernel(x, edge_index, edge_attr, pos, W1, b1, W2, b2, W3, b3):
    h = jnp.concatenate([x, pos], axis=1)

    pad = EPAD - EE
    src2 = jnp.concatenate(
        [edge_index[0], jnp.zeros((pad,), jnp.int32)]).reshape(NROW, CH)
    dst2 = jnp.concatenate(
        [edge_index[1], jnp.zeros((pad,), jnp.int32)]).reshape(NROW, CH)
    ea2 = jnp.concatenate(
        [edge_attr, jnp.zeros((pad,), jnp.float32)]).reshape(NROW, CH)

    norm2 = _norm_kernel(src2, dst2, ea2)

    srcP = src2.reshape(NROWP, CHP)
    dstP = dst2.reshape(NROWP, CHP)
    normP = norm2.reshape(NROWP, CHP)

    def layer(hin, W, b, fo, act):
        parts = _prop_kernel(hin, srcP, dstP, normP)
        p1, acc = _tc_first(hin, parts[0], parts[1], W[0], W[1], fo)
        parts = _prop_kernel(p1, srcP, dstP, normP)
        p2, acc = _tc_mid(acc, parts[0], parts[1], W[2], fo)
        parts = _prop_kernel(p2, srcP, dstP, normP)
        return _tc_last(acc, parts[0], parts[1], W[3], b, fo, act)

    h = layer(h, W1, b1, DD, "relu")
    h = layer(h, W2, b2, DD, "relu")
    return layer(h, W3, b3, 32, "lsm")


# speedup vs baseline: 1.0014x; 1.0014x over previous
import functools

import jax
import jax.numpy as jnp
import numpy as _np
from jax import lax
from jax.experimental import pallas as pl
from jax.experimental.pallas import tpu as pltpu
from jax.experimental.pallas import tpu_sc as plsc

NN = 10000
DD = 128
EE = 320000
NC = 2
NS = 16
NWK = NC * NS
CH = 128
EPAD = 327680
NROW = EPAD // CH
RPW = NROW // NWK
RPT = NROW // NS
NP = 10240
NPT = NP // NS
ROWS_PT = NN // NS

_mesh = plsc.VectorSubcoreMesh(core_axis_name="c", subcore_axis_name="s")


def _rsqrt_nr(x):
    i = plsc.bitcast(x, jnp.int32)
    i = jnp.int32(0x5F3759DF) - (i >> 1)
    y = plsc.bitcast(i, jnp.float32)
    for _ in range(4):
        y = y * (jnp.float32(1.5) - jnp.float32(0.5) * x * y * y)
    return y


@functools.partial(
    pl.kernel,
    out_type=jax.ShapeDtypeStruct((NROW, CH), jnp.float32),
    mesh=_mesh,
    scratch_types=[
        pltpu.VMEM((RPT, CH), jnp.int32),
        pltpu.VMEM((RPT, CH), jnp.float32),
        pltpu.VMEM((RPW, CH), jnp.int32),
        pltpu.VMEM((RPW, CH), jnp.float32),
        pltpu.VMEM((NP,), jnp.float32),
        pltpu.VMEM((NPT,), jnp.float32),
        pltpu.VMEM_SHARED((NP,), jnp.float32),
        pltpu.SemaphoreType.DMA,
    ],
    compiler_params=pltpu.CompilerParams(needs_layout_passes=False),
)
def _norm_kernel(src_hbm, dst_hbm, ea_hbm, norm_hbm,
                 dst_v, ea_v, src_v, norm_v, tab_v, z_v, deg_sh, dsem):
    c = lax.axis_index("c")
    s = lax.axis_index("s")
    wid = c * NS + s

    for i in range(NPT // 16):
        z_v[pl.ds(i * 16, 16)] = jnp.zeros((16,), jnp.float32)
    pltpu.sync_copy(z_v, deg_sh.at[pl.ds(s * NPT, NPT)])
    plsc.subcore_barrier()

    pltpu.sync_copy(dst_hbm.at[pl.ds(s * RPT, RPT)], dst_v)
    pltpu.sync_copy(ea_hbm.at[pl.ds(s * RPT, RPT)], ea_v)

    def deg_body(i, carry):
        for u in range(8):
            pltpu.async_copy(ea_v.at[i * 8 + u], deg_sh.at[dst_v.at[i * 8 + u]],
                             dsem, add=True)
        for u in range(8):
            pltpu.make_async_copy(ea_v.at[i * 8 + u],
                                  deg_sh.at[dst_v.at[i * 8 + u]], dsem).wait()
        return carry

    lax.fori_loop(0, RPT // 8, deg_body, 0)
    plsc.subcore_barrier()

    pltpu.sync_copy(deg_sh, tab_v)
    for i in range(NP // 16):
        d = tab_v[pl.ds(i * 16, 16)]
        ok = d > jnp.float32(0.0)
        y = _rsqrt_nr(jnp.where(ok, d, jnp.float32(1.0)))
        tab_v[pl.ds(i * 16, 16)] = jnp.where(ok, y, jnp.float32(0.0))

    pltpu.sync_copy(src_hbm.at[pl.ds(wid * RPW, RPW)], src_v)
    pltpu.sync_copy(dst_hbm.at[pl.ds(wid * RPW, RPW)], dst_v.at[pl.ds(0, RPW)])
    pltpu.sync_copy(ea_hbm.at[pl.ds(wid * RPW, RPW)], ea_v.at[pl.ds(0, RPW)])

    def norm_body(j, carry):
        for g in range(CH // 16):
            sl = pl.ds(g * 16, 16)
            s16 = src_v[j, sl]
            d16 = dst_v[j, sl]
            a16 = ea_v[j, sl]
            n16 = (plsc.load_gather(tab_v, [s16]) * a16
                   * plsc.load_gather(tab_v, [d16]))
            norm_v[j, sl] = n16
        return carry

    lax.fori_loop(0, RPW, norm_body, 0)
    pltpu.sync_copy(norm_v, norm_hbm.at[pl.ds(wid * RPW, RPW)])


_GDN = lax.GatherDimensionNumbers(offset_dims=(), collapsed_slice_dims=(0,),
                                  start_index_map=(0,))


def _splat(v, u):
    return lax.gather(v, jnp.full((16, 1), u, jnp.int32), _GDN, (1,),
                      mode=lax.GatherScatterMode.PROMISE_IN_BOUNDS)

NB = 4
CHP = 32
NROWP = EPAD // CHP
RPWP = NROWP // NWK
GG = 16
NGRP = RPWP // GG
ZCP = 640 // CHP


@functools.partial(
    pl.kernel,
    out_type=jax.ShapeDtypeStruct((NC, NN, DD), jnp.float32),
    mesh=_mesh,
    scratch_types=[
        pltpu.VMEM((2, GG, CHP), jnp.int32),
        pltpu.VMEM((2, GG, CHP), jnp.int32),
        pltpu.VMEM((2, GG, CHP), jnp.float32),
        pltpu.VMEM((NB, CHP, DD), jnp.float32),
        pltpu.VMEM_SHARED((NN, DD), jnp.float32),
        [pltpu.SemaphoreType.DMA] * 2,
        [pltpu.SemaphoreType.DMA] * NB,
        [pltpu.SemaphoreType.DMA] * NB,
    ],
    compiler_params=pltpu.CompilerParams(needs_layout_passes=False),
)
def _prop_kernel(h_hbm, src_hbm, dst_hbm, norm_hbm, out_hbm,
                 src_v, dst_v, norm_v, rows_v, acc_sh, esem, gsem, ssem):
    c = lax.axis_index("c")
    s = lax.axis_index("s")
    wid = c * NS + s
    base = wid * RPWP

    def stage_issue(grp, buf):
        rows = pl.ds(base + grp * GG, GG)
        pltpu.async_copy(src_hbm.at[rows], src_v.at[buf], esem[buf])
        pltpu.async_copy(dst_hbm.at[rows], dst_v.at[buf], esem[buf])
        pltpu.async_copy(norm_hbm.at[rows], norm_v.at[buf], esem[buf])

    def stage_wait(grp, buf):
        rows = pl.ds(base + grp * GG, GG)
        pltpu.make_async_copy(src_hbm.at[rows], src_v.at[buf],
                              esem[buf]).wait()
        pltpu.make_async_copy(dst_hbm.at[rows], dst_v.at[buf],
                              esem[buf]).wait()
        pltpu.make_async_copy(norm_hbm.at[rows], norm_v.at[buf],
                              esem[buf]).wait()

    def zero_body(r, zc):
        for g in range(DD // 16):
            rows_v[0, r, pl.ds(g * 16, 16)] = jnp.zeros((16,), jnp.float32)
        return zc

    lax.fori_loop(0, CHP, zero_body, 0)
    for t in range(ZCP):
        pltpu.sync_copy(rows_v.at[0],
                        acc_sh.at[pl.ds(s * 624 + t * CHP, CHP)])
    plsc.subcore_barrier()

    stage_issue(0, 0)
    stage_issue(1, 1)
    stage_wait(0, 0)
    for b in range(NB):
        pltpu.async_copy(h_hbm.at[src_v.at[0, b]], rows_v.at[b], gsem[b])

    def scale_chunk(b, gbuf, r):
        def half_body(i, ecarry):
            norm16 = norm_v[gbuf, r, pl.ds(i * 16, 16)]
            for u in range(16):
                e = i * 16 + u
                spl = _splat(norm16, u)
                for g in range(DD // 16):
                    sl = pl.ds(g * 16, 16)
                    rows_v[b, e, sl] = rows_v[b, e, sl] * spl
            return ecarry

        lax.fori_loop(0, CHP // 16, half_body, 0)

    def pair_body(i2, carry):
        for pp in range(2):
            grp = i2 * 2 + pp
            for r in range(GG):
                k = grp * GG + r
                b = r % NB

                if r == 1:
                    if pp == 0:
                        cond = i2 >= 1
                    else:
                        cond = i2 <= NGRP // 2 - 2

                    @pl.when(cond)
                    def _(grp=grp, pp=pp):
                        stage_issue(grp + 1, (pp + 1) % 2)

                if r == GG - NB:
                    cond = (i2 <= NGRP // 2 - 2) if pp == 1 else (i2 >= 0)

                    @pl.when(cond)
                    def _(grp=grp, pp=pp):
                        stage_wait(grp + 1, (pp + 1) % 2)

                pltpu.make_async_copy(h_hbm.at[src_v.at[pp, r]],
                                      rows_v.at[b], gsem[b]).wait()
                scale_chunk(b, pp, r)
                pltpu.async_copy(rows_v.at[b], acc_sh.at[dst_v.at[pp, r]],
                                 ssem[b], add=True)

                pb = (b - 1) % NB
                rn = (r + NB - 1) % GG
                pn = (pp + (1 if r + NB - 1 >= GG else 0)) % 2
                rm = (r - 1) % GG
                pm = (pp - (1 if r == 0 else 0)) % 2
                cond = jnp.logical_and(k >= 1, k <= RPWP - NB)

                @pl.when(cond)
                def _(pb=pb, rn=rn, pn=pn, rm=rm, pm=pm):
                    pltpu.make_async_copy(rows_v.at[pb],
                                          acc_sh.at[dst_v.at[pm, rm]],
                                          ssem[pb]).wait()
                    pltpu.async_copy(h_hbm.at[src_v.at[pn, rn]],
                                     rows_v.at[pb], gsem[pb])
        return carry

    lax.fori_loop(0, NGRP // 2, pair_body, 0)

    for b in range(NB):
        pltpu.make_async_copy(
            rows_v.at[b],
            acc_sh.at[dst_v.at[(NGRP - 1) % 2, GG - NB + b]],
            ssem[b]).wait()
    plsc.subcore_barrier()

    for t in range(ZCP):
        r0 = s * 624 + t * CHP
        pltpu.sync_copy(acc_sh.at[pl.ds(r0, CHP)], rows_v.at[0])
        pltpu.sync_copy(rows_v.at[0], out_hbm.at[c, pl.ds(r0, CHP)])


_BLK = 2000
_NBLK = NN // _BLK


def _vspec(d=DD):
    return pl.BlockSpec((_BLK, d), lambda i: (i, 0))


def _wspec(fi, fo):
    return pl.BlockSpec((fi, fo), lambda i: (0, 0))


def _tc_first(h, pa, pb, w0, w1, fo):
    def body(h_r, pa_r, pb_r, w0_r, w1_r, p_r, acc_r):
        p = pa_r[...] + pb_r[...]
        p_r[...] = p
        acc_r[...] = (jnp.dot(h_r[...], w0_r[...],
                              preferred_element_type=jnp.float32)
                      + jnp.dot(p, w1_r[...],
                                preferred_element_type=jnp.float32))

    return pl.pallas_call(
        body,
        grid=(_NBLK,),
        in_specs=[_vspec(), _vspec(), _vspec(), _wspec(DD, fo), _wspec(DD, fo)],
        out_specs=[_vspec(), _vspec(fo)],
        out_shape=[jax.ShapeDtypeStruct((NN, DD), jnp.float32),
                   jax.ShapeDtypeStruct((NN, fo), jnp.float32)],
    )(h, pa, pb, w0, w1)


def _tc_mid(acc, pa, pb, w, fo):
    def body(acc_r, pa_r, pb_r, w_r, p_r, out_r):
        p = pa_r[...] + pb_r[...]
        p_r[...] = p
        out_r[...] = acc_r[...] + jnp.dot(p, w_r[...],
                                          preferred_element_type=jnp.float32)

    return pl.pallas_call(
        body,
        grid=(_NBLK,),
        in_specs=[_vspec(fo), _vspec(), _vspec(), _wspec(DD, fo)],
        out_specs=[_vspec(), _vspec(fo)],
        out_shape=[jax.ShapeDtypeStruct((NN, DD), jnp.float32),
                   jax.ShapeDtypeStruct((NN, fo), jnp.float32)],
    )(acc, pa, pb, w)


def _tc_last(acc, pa, pb, w, b, fo, act):
    def body(acc_r, pa_r, pb_r, w_r, b_r, out_r):
        p = pa_r[...] + pb_r[...]
        o = acc_r[...] + jnp.dot(p, w_r[...],
                                 preferred_element_type=jnp.float32) + b_r[...]
        if act == "relu":
            out_r[...] = jnp.maximum(o, 0.0)
        else:
            m = jnp.max(o, axis=1, keepdims=True)
            z = o - m
            lse = jnp.log(jnp.sum(jnp.exp(z), axis=1, keepdims=True))
            out_r[...] = z - lse

    return pl.pallas_call(
        body,
        grid=(_NBLK,),
        in_specs=[_vspec(fo), _vspec(), _vspec(), _wspec(DD, fo),
                  pl.BlockSpec((1, fo), lambda i: (0, 0))],
        out_specs=_vspec(fo),
        out_shape=jax.ShapeDtypeStruct((NN, fo), jnp.float32),
    )(acc, pa, pb, w, b.reshape(1, fo))


def kernel(x, edge_index, edge_attr, pos, W1, b1, W2, b2, W3, b3):
    h = jnp.concatenate([x, pos], axis=1)

    pad = EPAD - EE
    src2 = jnp.concatenate(
        [edge_index[0], jnp.zeros((pad,), jnp.int32)]).reshape(NROW, CH)
    dst2 = jnp.concatenate(
        [edge_index[1], jnp.zeros((pad,), jnp.int32)]).reshape(NROW, CH)
    ea2 = jnp.concatenate(
        [edge_attr, jnp.zeros((pad,), jnp.float32)]).reshape(NROW, CH)

    norm2 = _norm_kernel(src2, dst2, ea2)

    srcP = src2.reshape(NROWP, CHP)
    dstP = dst2.reshape(NROWP, CHP)
    normP = norm2.reshape(NROWP, CHP)

    def layer(hin, W, b, fo, act):
        parts = _prop_kernel(hin, srcP, dstP, normP)
        p1, acc = _tc_first(hin, parts[0], parts[1], W[0], W[1], fo)
        parts = _prop_kernel(p1, srcP, dstP, normP)
        p2, acc = _tc_mid(acc, parts[0], parts[1], W[2], fo)
        parts = _prop_kernel(p2, srcP, dstP, normP)
        return _tc_last(acc, parts[0], parts[1], W[3], b, fo, act)

    h = layer(h, W1, b1, DD, "relu")
    h = layer(h, W2, b2, DD, "relu")
    return layer(h, W3, b3, 32, "lsm")
